# Initial kernel scaffold; baseline (speedup 1.0000x reference)
import jax
import jax.numpy as jnp
from jax.experimental import pallas as pl


def kernel(features, points, W0, b0, gamma0, beta0, W1, b1, gamma1, beta1, W2, b2, gamma2, beta2):
    raise NotImplementedError("write your pallas kernel here")


# speedup vs baseline: 79.7325x; 79.7325x over previous
import functools

import jax
import jax.numpy as jnp
import numpy as np
from jax import lax
from jax.experimental import pallas as pl
from jax.experimental.pallas import tpu as pltpu
from jax.experimental.pallas import tpu_sc as plsc

B, N, F = 4, 4096, 64
S, K = 1024, 32
R2 = np.float32(0.4 * 0.4)
EPS = np.float32(1e-5)
P = B * S * K
CIN = 80
NW = 32
ROWS_PER_W = P // NW
CHUNK = 128
NCHUNK = ROWS_PER_W // CHUNK


def _fps_body(pts_ref, out_ref):
    arr = pts_ref[...]
    x = arr[:, 0, :]
    y = arr[:, 1, :]
    z = arr[:, 2, :]
    iota = lax.broadcasted_iota(jnp.int32, (B, N), 1)

    def body(i, carry):
        dists, far = carry
        sel = iota == far
        cx = jnp.sum(jnp.where(sel, x, 0.0), axis=1, keepdims=True)
        cy = jnp.sum(jnp.where(sel, y, 0.0), axis=1, keepdims=True)
        cz = jnp.sum(jnp.where(sel, z, 0.0), axis=1, keepdims=True)
        row = jnp.concatenate([cx, cy, cz], axis=1)
        out_ref[pl.ds(i, 1), :, :] = row[None]
        dx = x - cx
        dy = y - cy
        dz = z - cz
        d = (dx * dx + dy * dy) + dz * dz
        dists = jnp.minimum(dists, d)
        m = jnp.max(dists, axis=1, keepdims=True)
        cand = jnp.where(dists == m, iota, N)
        far = jnp.min(cand, axis=1, keepdims=True)
        return dists, far

    dists0 = jnp.full((B, N), 1e10, dtype=jnp.float32)
    far0 = jnp.zeros((B, 1), dtype=jnp.int32)
    lax.fori_loop(0, S, body, (dists0, far0))


def _fps(pts):
    return pl.pallas_call(
        _fps_body,
        out_shape=jax.ShapeDtypeStruct((S, B, 3), jnp.float32),
    )(pts)


S_T = 128


def _bq_body(cents_ref, pts_ref, out_ref):
    c = cents_ref[0]
    p = pts_ref[0]
    x = p[0:1, :]
    y = p[1:2, :]
    z = p[2:3, :]
    cx = c[:, 0:1]
    cy = c[:, 1:2]
    cz = c[:, 2:3]
    an = (cx * cx + cy * cy) + cz * cz
    bn = (x * x + y * y) + z * z
    rnd = lambda v: v.astype(jnp.bfloat16).astype(jnp.float32)
    cross = (rnd(cx) * rnd(x) + rnd(cy) * rnd(y)) + rnd(cz) * rnd(z)
    d2 = (an + bn) - 2.0 * cross
    iota = lax.broadcasted_iota(jnp.int32, (S_T, N), 1)
    key = jnp.where(d2 <= R2, iota, N)
    cols = []
    for _ in range(K):
        m = jnp.min(key, axis=1, keepdims=True)
        cols.append(jnp.minimum(m, N))
        key = jnp.where(key == m, N + 1, key)
    idx = jnp.concatenate(cols, axis=1)
    first = idx[:, 0:1]
    idx = jnp.where(idx >= N, first, idx)
    b = pl.program_id(0)
    out_ref[0] = idx + b * N


def _ball_query(cents, pts):
    return pl.pallas_call(
        _bq_body,
        grid=(B, S // S_T),
        in_specs=[
            pl.BlockSpec((1, S_T, 3), lambda b, sb: (b, sb, 0)),
            pl.BlockSpec((1, 3, N), lambda b, sb: (b, 0, 0)),
        ],
        out_specs=pl.BlockSpec((1, S_T, K), lambda b, sb: (b, sb, 0)),
        out_shape=jax.ShapeDtypeStruct((B, S, K), jnp.int32),
    )(cents, pts)


def _sc_gather_body(table_hbm, idx_hbm, out_hbm, idx_v, rows_a, rows_b,
                    gsem, ssem_a, ssem_b):
    wid = lax.axis_index("c") * 16 + lax.axis_index("s")
    base = wid * ROWS_PER_W
    pltpu.sync_copy(idx_hbm.at[wid], idx_v)

    def body(jj, carry):
        j0 = jj * 2
        pltpu.async_copy(table_hbm.at[idx_v.at[j0]], rows_a, gsem).wait()
        sa = pltpu.async_copy(
            rows_a, out_hbm.at[pl.ds(base + j0 * CHUNK, CHUNK)], ssem_a)
        pltpu.async_copy(table_hbm.at[idx_v.at[j0 + 1]], rows_b, gsem).wait()
        sb = pltpu.async_copy(
            rows_b, out_hbm.at[pl.ds(base + (j0 + 1) * CHUNK, CHUNK)], ssem_b)
        sa.wait()
        sb.wait()
        return carry

    lax.fori_loop(0, NCHUNK // 2, body, 0)


def _sc_gather(table, gidx):
    mesh = plsc.VectorSubcoreMesh(core_axis_name="c", subcore_axis_name="s")
    return pl.kernel(
        _sc_gather_body,
        out_type=jax.ShapeDtypeStruct((P, CIN), jnp.float32),
        mesh=mesh,
        scratch_types=[
            pltpu.VMEM((NCHUNK, CHUNK), jnp.int32),
            pltpu.VMEM((CHUNK, CIN), jnp.float32),
            pltpu.VMEM((CHUNK, CIN), jnp.float32),
            pltpu.SemaphoreType.DMA,
            pltpu.SemaphoreType.DMA,
            pltpu.SemaphoreType.DMA,
        ],
        compiler_params=pltpu.CompilerParams(use_tc_tiling_on_sc=False),
    )(table, gidx)


P_T = 2048
G_T = P_T // K
NSTEP = P // P_T
INV_P = np.float32(1.0 / P)


def _mlp0_body(x_ref, c_ref, w_ref, b_ref, h_ref, st_ref):
    w = w_ref[...]
    h = jnp.dot(x_ref[...], w, preferred_element_type=jnp.float32)
    c = c_ref[...]
    corr = (c[:, 0:1] * w[0:1, :] + c[:, 1:2] * w[1:2, :]
            + c[:, 2:3] * w[2:3, :])
    corr_full = jnp.broadcast_to(
        corr[:, None, :], (G_T, K, corr.shape[-1])).reshape(P_T, -1)
    h = (h + b_ref[...]) - corr_full
    h_ref[...] = h

    @pl.when(pl.program_id(0) == 0)
    def _():
        st_ref[...] = jnp.zeros_like(st_ref)

    st_ref[0:1, :] += jnp.sum(h, axis=0, keepdims=True)
    st_ref[1:2, :] += jnp.sum(h * h, axis=0, keepdims=True)


def _mlp0(x, cents_flat, w0t, b0):
    cout = w0t.shape[1]
    return pl.pallas_call(
        _mlp0_body,
        grid=(NSTEP,),
        in_specs=[
            pl.BlockSpec((P_T, CIN), lambda i: (i, 0)),
            pl.BlockSpec((G_T, 3), lambda i: (i, 0)),
            pl.BlockSpec((CIN, cout), lambda i: (0, 0)),
            pl.BlockSpec((1, cout), lambda i: (0, 0)),
        ],
        out_specs=[
            pl.BlockSpec((P_T, cout), lambda i: (i, 0)),
            pl.BlockSpec((8, cout), lambda i: (0, 0)),
        ],
        out_shape=[
            jax.ShapeDtypeStruct((P, cout), jnp.float32),
            jax.ShapeDtypeStruct((8, cout), jnp.float32),
        ],
    )(x, cents_flat, w0t, b0)


def _norm_act(h, st_ref, g_ref, be_ref):
    mean = st_ref[0:1, :] * INV_P
    var = st_ref[1:2, :] * INV_P - mean * mean
    hn = (h - mean) / jnp.sqrt(var + EPS) * g_ref[...] + be_ref[...]
    return jnp.where(hn >= 0, hn, 0.2 * hn)


def _mlp_mid_body(h_ref, st_in_ref, g_ref, be_ref, w_ref, b_ref,
                  o_ref, st_ref):
    act = _norm_act(h_ref[...], st_in_ref, g_ref, be_ref)
    h = jnp.dot(act, w_ref[...], preferred_element_type=jnp.float32)
    h = h + b_ref[...]
    o_ref[...] = h

    @pl.when(pl.program_id(0) == 0)
    def _():
        st_ref[...] = jnp.zeros_like(st_ref)

    st_ref[0:1, :] += jnp.sum(h, axis=0, keepdims=True)
    st_ref[1:2, :] += jnp.sum(h * h, axis=0, keepdims=True)


def _mlp_mid(h, st, g, be, wt, b):
    cin, cout = wt.shape
    return pl.pallas_call(
        _mlp_mid_body,
        grid=(NSTEP,),
        in_specs=[
            pl.BlockSpec((P_T, cin), lambda i: (i, 0)),
            pl.BlockSpec((8, cin), lambda i: (0, 0)),
            pl.BlockSpec((1, cin), lambda i: (0, 0)),
            pl.BlockSpec((1, cin), lambda i: (0, 0)),
            pl.BlockSpec((cin, cout), lambda i: (0, 0)),
            pl.BlockSpec((1, cout), lambda i: (0, 0)),
        ],
        out_specs=[
            pl.BlockSpec((P_T, cout), lambda i: (i, 0)),
            pl.BlockSpec((8, cout), lambda i: (0, 0)),
        ],
        out_shape=[
            jax.ShapeDtypeStruct((P, cout), jnp.float32),
            jax.ShapeDtypeStruct((8, cout), jnp.float32),
        ],
    )(h, st, g, be, wt, b)


def _fin_body(h_ref, st_in_ref, g_ref, be_ref, o_ref):
    act = _norm_act(h_ref[...], st_in_ref, g_ref, be_ref)
    cout = act.shape[-1]
    o_ref[...] = jnp.max(act.reshape(G_T, K, cout), axis=1)


def _mlp_fin(h, st, g, be):
    cin = h.shape[1]
    return pl.pallas_call(
        _fin_body,
        grid=(NSTEP,),
        in_specs=[
            pl.BlockSpec((P_T, cin), lambda i: (i, 0)),
            pl.BlockSpec((8, cin), lambda i: (0, 0)),
            pl.BlockSpec((1, cin), lambda i: (0, 0)),
            pl.BlockSpec((1, cin), lambda i: (0, 0)),
        ],
        out_specs=pl.BlockSpec((G_T, cin), lambda i: (i, 0)),
        out_shape=jax.ShapeDtypeStruct((B * S, cin), jnp.float32),
    )(h, st, g, be)


def kernel(features, points, W0, b0, gamma0, beta0, W1, b1, gamma1, beta1,
           W2, b2, gamma2, beta2):
    pts = points.astype(jnp.float32)

    cent_rows = _fps(pts)
    cents = jnp.transpose(cent_rows, (1, 0, 2))
    new_xyz = jnp.transpose(cent_rows, (1, 2, 0))

    gidx = _ball_query(cents, pts)

    table = jnp.concatenate(
        [jnp.transpose(pts, (0, 2, 1)),
         jnp.transpose(features, (0, 2, 1)),
         jnp.zeros((B, N, CIN - 3 - F), jnp.float32)],
        axis=-1).reshape(B * N, CIN)
    x = _sc_gather(table, gidx.reshape(NW, NCHUNK, CHUNK))

    w0t = jnp.pad(W0, ((0, 0), (0, CIN - 3 - F))).T
    cents_flat = cents.reshape(B * S, 3)
    h0, st0 = _mlp0(x, cents_flat, w0t, b0.reshape(1, -1))
    h1, st1 = _mlp_mid(h0, st0, gamma0.reshape(1, -1), beta0.reshape(1, -1),
                       W1.T, b1.reshape(1, -1))
    h2, st2 = _mlp_mid(h1, st1, gamma1.reshape(1, -1), beta1.reshape(1, -1),
                       W2.T, b2.reshape(1, -1))
    pooled = _mlp_fin(h2, st2, gamma2.reshape(1, -1), beta2.reshape(1, -1))

    new_features = jnp.transpose(pooled.reshape(B, S, -1), (0, 2, 1))
    return (new_features, new_xyz)


# speedup vs baseline: 100.5482x; 1.2611x over previous
import functools

import jax
import jax.numpy as jnp
import numpy as np
from jax import lax
from jax.experimental import pallas as pl
from jax.experimental.pallas import tpu as pltpu
from jax.experimental.pallas import tpu_sc as plsc

B, N, F = 4, 4096, 64
S, K = 1024, 32
R2 = np.float32(0.4 * 0.4)
EPS = np.float32(1e-5)
P = B * S * K
CIN = 80
NW = 32
ROWS_PER_W = P // NW
CHUNK = 128
NCHUNK = ROWS_PER_W // CHUNK


FSUB, FLANE = 8, N // 8


def _fps_body(pts_ref, out_ref):
    arr = pts_ref[...]
    x = arr[:, 0]
    y = arr[:, 1]
    z = arr[:, 2]
    iota = (lax.broadcasted_iota(jnp.int32, (B, FSUB, FLANE), 1) * FLANE
            + lax.broadcasted_iota(jnp.int32, (B, FSUB, FLANE), 2))

    def body(i, carry):
        dists, far = carry
        sel = iota == far
        cx = jnp.sum(jnp.where(sel, x, 0.0), axis=(1, 2), keepdims=True)
        cy = jnp.sum(jnp.where(sel, y, 0.0), axis=(1, 2), keepdims=True)
        cz = jnp.sum(jnp.where(sel, z, 0.0), axis=(1, 2), keepdims=True)
        row = jnp.concatenate([cx, cy, cz], axis=2)
        out_ref[pl.ds(i, 1), :, :] = row.reshape(1, B, 3)
        dx = x - cx
        dy = y - cy
        dz = z - cz
        d = (dx * dx + dy * dy) + dz * dz
        dists = jnp.minimum(dists, d)
        m = jnp.max(dists, axis=(1, 2), keepdims=True)
        cand = jnp.where(dists == m, iota, N)
        far = jnp.min(cand, axis=(1, 2), keepdims=True)
        return dists, far

    dists0 = jnp.full((B, FSUB, FLANE), 1e10, dtype=jnp.float32)
    far0 = jnp.zeros((B, 1, 1), dtype=jnp.int32)
    lax.fori_loop(0, S, body, (dists0, far0))


def _fps(pts):
    return pl.pallas_call(
        _fps_body,
        out_shape=jax.ShapeDtypeStruct((S, B, 3), jnp.float32),
    )(pts.reshape(B, 3, FSUB, FLANE))


S_T = 128


def _bq_body(cents_ref, pts_ref, out_ref):
    c = cents_ref[0]
    p = pts_ref[0]
    x = p[0:1, :]
    y = p[1:2, :]
    z = p[2:3, :]
    cx = c[:, 0:1]
    cy = c[:, 1:2]
    cz = c[:, 2:3]
    an = (cx * cx + cy * cy) + cz * cz
    bn = (x * x + y * y) + z * z
    rnd = lambda v: v.astype(jnp.bfloat16).astype(jnp.float32)
    cross = (rnd(cx) * rnd(x) + rnd(cy) * rnd(y)) + rnd(cz) * rnd(z)
    d2 = (an + bn) - 2.0 * cross
    iota = lax.broadcasted_iota(jnp.int32, (S_T, N), 1)
    key = jnp.where(d2 <= R2, iota, N)
    cols = []
    for _ in range(K):
        m = jnp.min(key, axis=1, keepdims=True)
        cols.append(jnp.minimum(m, N))
        key = jnp.where(key == m, N + 1, key)
    idx = jnp.concatenate(cols, axis=1)
    first = idx[:, 0:1]
    idx = jnp.where(idx >= N, first, idx)
    b = pl.program_id(0)
    out_ref[0] = idx + b * N


def _ball_query(cents, pts):
    return pl.pallas_call(
        _bq_body,
        grid=(B, S // S_T),
        in_specs=[
            pl.BlockSpec((1, S_T, 3), lambda b, sb: (b, sb, 0)),
            pl.BlockSpec((1, 3, N), lambda b, sb: (b, 0, 0)),
        ],
        out_specs=pl.BlockSpec((1, S_T, K), lambda b, sb: (b, sb, 0)),
        out_shape=jax.ShapeDtypeStruct((B, S, K), jnp.int32),
    )(cents, pts)


def _sc_gather_body(table_hbm, idx_hbm, out_hbm, idx_v, rows_a, rows_b,
                    gsem, ssem_a, ssem_b):
    wid = lax.axis_index("c") * 16 + lax.axis_index("s")
    base = wid * ROWS_PER_W
    pltpu.sync_copy(idx_hbm.at[wid], idx_v)

    def body(jj, carry):
        j0 = jj * 2
        pltpu.async_copy(table_hbm.at[idx_v.at[j0]], rows_a, gsem).wait()
        sa = pltpu.async_copy(
            rows_a, out_hbm.at[pl.ds(base + j0 * CHUNK, CHUNK)], ssem_a)
        pltpu.async_copy(table_hbm.at[idx_v.at[j0 + 1]], rows_b, gsem).wait()
        sb = pltpu.async_copy(
            rows_b, out_hbm.at[pl.ds(base + (j0 + 1) * CHUNK, CHUNK)], ssem_b)
        sa.wait()
        sb.wait()
        return carry

    lax.fori_loop(0, NCHUNK // 2, body, 0)


def _sc_gather(table, gidx):
    mesh = plsc.VectorSubcoreMesh(core_axis_name="c", subcore_axis_name="s")
    return pl.kernel(
        _sc_gather_body,
        out_type=jax.ShapeDtypeStruct((P, CIN), jnp.float32),
        mesh=mesh,
        scratch_types=[
            pltpu.VMEM((NCHUNK, CHUNK), jnp.int32),
            pltpu.VMEM((CHUNK, CIN), jnp.float32),
            pltpu.VMEM((CHUNK, CIN), jnp.float32),
            pltpu.SemaphoreType.DMA,
            pltpu.SemaphoreType.DMA,
            pltpu.SemaphoreType.DMA,
        ],
        compiler_params=pltpu.CompilerParams(use_tc_tiling_on_sc=False),
    )(table, gidx)


P_T = 2048
G_T = P_T // K
NSTEP = P // P_T
INV_P = np.float32(1.0 / P)


def _mlp0_body(x_ref, c_ref, w_ref, b_ref, h_ref, st_ref):
    w = w_ref[...]
    h = jnp.dot(x_ref[...], w, preferred_element_type=jnp.float32)
    c = c_ref[...]
    corr = (c[:, 0:1] * w[0:1, :] + c[:, 1:2] * w[1:2, :]
            + c[:, 2:3] * w[2:3, :])
    corr_full = jnp.broadcast_to(
        corr[:, None, :], (G_T, K, corr.shape[-1])).reshape(P_T, -1)
    h = (h + b_ref[...]) - corr_full
    h_ref[...] = h

    @pl.when(pl.program_id(0) == 0)
    def _():
        st_ref[...] = jnp.zeros_like(st_ref)

    st_ref[0:1, :] += jnp.sum(h, axis=0, keepdims=True)
    st_ref[1:2, :] += jnp.sum(h * h, axis=0, keepdims=True)


def _mlp0(x, cents_flat, w0t, b0):
    cout = w0t.shape[1]
    return pl.pallas_call(
        _mlp0_body,
        grid=(NSTEP,),
        in_specs=[
            pl.BlockSpec((P_T, CIN), lambda i: (i, 0)),
            pl.BlockSpec((G_T, 3), lambda i: (i, 0)),
            pl.BlockSpec((CIN, cout), lambda i: (0, 0)),
            pl.BlockSpec((1, cout), lambda i: (0, 0)),
        ],
        out_specs=[
            pl.BlockSpec((P_T, cout), lambda i: (i, 0)),
            pl.BlockSpec((8, cout), lambda i: (0, 0)),
        ],
        out_shape=[
            jax.ShapeDtypeStruct((P, cout), jnp.float32),
            jax.ShapeDtypeStruct((8, cout), jnp.float32),
        ],
    )(x, cents_flat, w0t, b0)


def _norm_act(h, st_ref, g_ref, be_ref):
    mean = st_ref[0:1, :] * INV_P
    var = st_ref[1:2, :] * INV_P - mean * mean
    hn = (h - mean) / jnp.sqrt(var + EPS) * g_ref[...] + be_ref[...]
    return jnp.where(hn >= 0, hn, 0.2 * hn)


def _mlp_mid_body(h_ref, st_in_ref, g_ref, be_ref, w_ref, b_ref,
                  o_ref, st_ref):
    act = _norm_act(h_ref[...], st_in_ref, g_ref, be_ref)
    h = jnp.dot(act, w_ref[...], preferred_element_type=jnp.float32)
    h = h + b_ref[...]
    o_ref[...] = h

    @pl.when(pl.program_id(0) == 0)
    def _():
        st_ref[...] = jnp.zeros_like(st_ref)

    st_ref[0:1, :] += jnp.sum(h, axis=0, keepdims=True)
    st_ref[1:2, :] += jnp.sum(h * h, axis=0, keepdims=True)


def _mlp_mid(h, st, g, be, wt, b):
    cin, cout = wt.shape
    return pl.pallas_call(
        _mlp_mid_body,
        grid=(NSTEP,),
        in_specs=[
            pl.BlockSpec((P_T, cin), lambda i: (i, 0)),
            pl.BlockSpec((8, cin), lambda i: (0, 0)),
            pl.BlockSpec((1, cin), lambda i: (0, 0)),
            pl.BlockSpec((1, cin), lambda i: (0, 0)),
            pl.BlockSpec((cin, cout), lambda i: (0, 0)),
            pl.BlockSpec((1, cout), lambda i: (0, 0)),
        ],
        out_specs=[
            pl.BlockSpec((P_T, cout), lambda i: (i, 0)),
            pl.BlockSpec((8, cout), lambda i: (0, 0)),
        ],
        out_shape=[
            jax.ShapeDtypeStruct((P, cout), jnp.float32),
            jax.ShapeDtypeStruct((8, cout), jnp.float32),
        ],
    )(h, st, g, be, wt, b)


def _fin_body(h_ref, st_in_ref, g_ref, be_ref, o_ref):
    act = _norm_act(h_ref[...], st_in_ref, g_ref, be_ref)
    cout = act.shape[-1]
    o_ref[...] = jnp.max(act.reshape(G_T, K, cout), axis=1)


def _mlp_fin(h, st, g, be):
    cin = h.shape[1]
    return pl.pallas_call(
        _fin_body,
        grid=(NSTEP,),
        in_specs=[
            pl.BlockSpec((P_T, cin), lambda i: (i, 0)),
            pl.BlockSpec((8, cin), lambda i: (0, 0)),
            pl.BlockSpec((1, cin), lambda i: (0, 0)),
            pl.BlockSpec((1, cin), lambda i: (0, 0)),
        ],
        out_specs=pl.BlockSpec((G_T, cin), lambda i: (i, 0)),
        out_shape=jax.ShapeDtypeStruct((B * S, cin), jnp.float32),
    )(h, st, g, be)


def kernel(features, points, W0, b0, gamma0, beta0, W1, b1, gamma1, beta1,
           W2, b2, gamma2, beta2):
    pts = points.astype(jnp.float32)

    cent_rows = _fps(pts)
    cents = jnp.transpose(cent_rows, (1, 0, 2))
    new_xyz = jnp.transpose(cent_rows, (1, 2, 0))

    gidx = _ball_query(cents, pts)

    table = jnp.concatenate(
        [jnp.transpose(pts, (0, 2, 1)),
         jnp.transpose(features, (0, 2, 1)),
         jnp.zeros((B, N, CIN - 3 - F), jnp.float32)],
        axis=-1).reshape(B * N, CIN)
    x = _sc_gather(table, gidx.reshape(NW, NCHUNK, CHUNK))

    w0t = jnp.pad(W0, ((0, 0), (0, CIN - 3 - F))).T
    cents_flat = cents.reshape(B * S, 3)
    h0, st0 = _mlp0(x, cents_flat, w0t, b0.reshape(1, -1))
    h1, st1 = _mlp_mid(h0, st0, gamma0.reshape(1, -1), beta0.reshape(1, -1),
                       W1.T, b1.reshape(1, -1))
    h2, st2 = _mlp_mid(h1, st1, gamma1.reshape(1, -1), beta1.reshape(1, -1),
                       W2.T, b2.reshape(1, -1))
    pooled = _mlp_fin(h2, st2, gamma2.reshape(1, -1), beta2.reshape(1, -1))

    new_features = jnp.transpose(pooled.reshape(B, S, -1), (0, 2, 1))
    return (new_features, new_xyz)
